# baseline (speedup 1.0000x reference)
import functools

import jax
import jax.numpy as jnp
from jax import lax
from jax.experimental import pallas as pl
from jax.experimental.pallas import tpu as pltpu
from jax.experimental.pallas import tpu_sc as plsc

DIM = 64
LANES = 16
IDX_CHUNK = 128
HALF = 256


@functools.lru_cache(maxsize=None)
def _make_kernel(batch: int, vocab: int):
    info = plsc.get_sparse_core_info()
    nc, ns = info.num_cores, info.num_subcores
    nw = nc * ns
    bpw = batch // nw
    nch = HALF // IDX_CHUNK
    mesh = plsc.VectorSubcoreMesh(core_axis_name="c", subcore_axis_name="s")

    @functools.partial(
        pl.kernel,
        mesh=mesh,
        out_type=jax.ShapeDtypeStruct((batch,), jnp.float32),
        scratch_types=[
            pltpu.VMEM((bpw,), jnp.int32),
            pltpu.VMEM((bpw,), jnp.int32),
            pltpu.VMEM((nch, IDX_CHUNK), jnp.int32),
            pltpu.VMEM((nch, IDX_CHUNK), jnp.int32),
            pltpu.VMEM((HALF, 2 * DIM), jnp.float32),
            pltpu.VMEM((HALF, 2 * DIM), jnp.float32),
            pltpu.VMEM((bpw,), jnp.float32),
            pltpu.SemaphoreType.DMA,
        ],
        compiler_params=pltpu.CompilerParams(needs_layout_passes=False),
    )
    def skipgram(center_hbm, context_hbm, in2_hbm, out2_hbm, o_hbm,
                 cv, xv, crow, xrow, abuf, cbuf, ovec, sem):
        wid = lax.axis_index("s") * nc + lax.axis_index("c")
        base = wid * bpw

        pltpu.sync_copy(center_hbm.at[pl.ds(base, bpw)], cv)
        pltpu.sync_copy(context_hbm.at[pl.ds(base, bpw)], xv)

        lane = lax.iota(jnp.int32, LANES)

        def half_pass(h, carry):
            off = pl.multiple_of(h * HALF, HALF)

            def shift_body(t, carry2):
                c = t // (IDX_CHUNK // LANES)
                w = t % (IDX_CHUNK // LANES)
                dst = pl.ds(pl.multiple_of(w * LANES, LANES), LANES)
                src = pl.ds(off + pl.multiple_of(t * LANES, LANES), LANES)
                crow[c, dst] = cv[src] >> 1
                xrow[c, dst] = xv[src] >> 1
                return carry2

            lax.fori_loop(0, HALF // LANES, shift_body, 0)

            copies = []
            for c in range(nch):
                copies.append(pltpu.async_copy(
                    in2_hbm.at[crow.at[c]],
                    abuf.at[pl.ds(c * IDX_CHUNK, IDX_CHUNK)], sem))
                copies.append(pltpu.async_copy(
                    out2_hbm.at[xrow.at[c]],
                    cbuf.at[pl.ds(c * IDX_CHUNK, IDX_CHUNK)], sem))
            for cp in copies:
                cp.wait()

            def score_group(g, carry2):
                row0 = g * LANES
                accs = [jnp.zeros((LANES,), jnp.float32) for _ in range(4)]
                for r in range(LANES):
                    i = row0 + r
                    av = [abuf[i, pl.ds(k * LANES, LANES)] for k in range(8)]
                    xw = [cbuf[i, pl.ds(k * LANES, LANES)] for k in range(8)]
                    m = lane == r
                    for combo in range(4):
                        ao = (combo >> 1) * 4
                        co = (combo & 1) * 4
                        s = av[ao] * xw[co]
                        for k in range(1, 4):
                            s = s + av[ao + k] * xw[co + k]
                        accs[combo] = jnp.where(m, jnp.sum(s), accs[combo])
                sl = pl.ds(pl.multiple_of(off + row0, LANES), LANES)
                pa = cv[sl] & 1
                pc = xv[sl] & 1
                ovec[sl] = jnp.where(
                    pa == 0,
                    jnp.where(pc == 0, accs[0], accs[1]),
                    jnp.where(pc == 0, accs[2], accs[3]),
                )
                return carry2

            lax.fori_loop(0, HALF // LANES, score_group, 0)
            return carry

        lax.fori_loop(0, bpw // HALF, half_pass, 0)
        pltpu.sync_copy(ovec, o_hbm.at[pl.ds(base, bpw)])

    return skipgram


def kernel(center_words, context_words, in_emb, out_emb):
    (batch,) = center_words.shape
    vocab, dim = in_emb.shape
    in2 = jnp.concatenate([in_emb[0::2], in_emb[1::2]], axis=1)
    out2 = jnp.concatenate([out_emb[0::2], out_emb[1::2]], axis=1)
    return _make_kernel(batch, vocab)(center_words, context_words, in2, out2)


# speedup vs baseline: 15.8404x; 15.8404x over previous
import functools

import jax
import jax.numpy as jnp
from jax import lax
from jax.experimental import pallas as pl
from jax.experimental.pallas import tpu as pltpu
from jax.experimental.pallas import tpu_sc as plsc

DIM = 64
LANES = 16
IDX_CHUNK = 128


@functools.lru_cache(maxsize=None)
def _make_kernel(batch: int):
    info = plsc.get_sparse_core_info()
    nc, ns = info.num_cores, info.num_subcores
    nw = nc * ns
    bpw = batch // nw
    nch = bpw // IDX_CHUNK
    mesh = plsc.VectorSubcoreMesh(core_axis_name="c", subcore_axis_name="s")

    @functools.partial(
        pl.kernel,
        mesh=mesh,
        out_type=jax.ShapeDtypeStruct((batch,), jnp.float32),
        scratch_types=[
            pltpu.VMEM((nch, IDX_CHUNK), jnp.int32),
            pltpu.VMEM((nch, IDX_CHUNK), jnp.int32),
            pltpu.VMEM((bpw, DIM), jnp.float32),
            pltpu.VMEM((bpw, DIM), jnp.float32),
            pltpu.VMEM((bpw,), jnp.float32),
            pltpu.SemaphoreType.DMA,
        ],
        compiler_params=pltpu.CompilerParams(
            needs_layout_passes=False, use_tc_tiling_on_sc=False),
    )
    def skipgram(center_hbm, context_hbm, inemb_hbm, outemb_hbm, o_hbm,
                 cidx, xidx, arows, crows, ovec, sem):
        wid = lax.axis_index("s") * nc + lax.axis_index("c")
        base = wid * bpw

        for j in range(nch):
            pltpu.sync_copy(center_hbm.at[pl.ds(base + j * IDX_CHUNK, IDX_CHUNK)],
                            cidx.at[j])
            pltpu.sync_copy(context_hbm.at[pl.ds(base + j * IDX_CHUNK, IDX_CHUNK)],
                            xidx.at[j])

        copies = []
        for j in range(nch):
            copies.append(pltpu.async_copy(
                inemb_hbm.at[cidx.at[j]],
                arows.at[pl.ds(j * IDX_CHUNK, IDX_CHUNK)], sem))
            copies.append(pltpu.async_copy(
                outemb_hbm.at[xidx.at[j]],
                crows.at[pl.ds(j * IDX_CHUNK, IDX_CHUNK)], sem))
        for cp in copies:
            cp.wait()

        lane = lax.iota(jnp.int32, LANES)

        def group_body(g, carry):
            row0 = g * LANES
            acc = jnp.zeros((LANES,), jnp.float32)
            for r in range(LANES):
                row = row0 + r
                s = arows[row, pl.ds(0, LANES)] * crows[row, pl.ds(0, LANES)]
                for k in range(1, DIM // LANES):
                    s = s + (arows[row, pl.ds(k * LANES, LANES)]
                             * crows[row, pl.ds(k * LANES, LANES)])
                acc = jnp.where(lane == r, jnp.sum(s), acc)
            ovec[pl.ds(pl.multiple_of(row0, LANES), LANES)] = acc
            return carry

        lax.fori_loop(0, bpw // LANES, group_body, 0)
        pltpu.sync_copy(ovec, o_hbm.at[pl.ds(base, bpw)])

    return skipgram


def kernel(center_words, context_words, in_emb, out_emb):
    (batch,) = center_words.shape
    return _make_kernel(batch)(center_words, context_words, in_emb, out_emb)
